# baseline (speedup 1.0000x reference)
import jax
import jax.numpy as jnp
from jax.experimental import pallas as pl
from jax.experimental.pallas import tpu as pltpu

H = 224
W = 224
WP = 256
CIN = 192
CMID = 256
R = 16
T = H // R
M = R * WP
SEG = (R + 2) * WP + 128


def _rpn_body(a_ref, b_ref, w_ref, wcr_ref, cb_ref, crb_ref,
              cls_ref, reg_ref, seg_ref, top_ref):
    i = pl.program_id(0)

    @pl.when(i == 0)
    def _():
        seg_ref[...] = jnp.zeros((2, CIN, SEG), jnp.bfloat16)

    cen = jnp.transpose(a_ref[...].astype(jnp.bfloat16), (1, 0, 2))
    nxt = b_ref[:, 0, :].astype(jnp.bfloat16)
    top = jnp.where(i == 0, jnp.zeros_like(top_ref), top_ref[...])
    nxt = jnp.where(i == T - 1, jnp.zeros_like(nxt), nxt)
    b = jax.lax.rem(i, 2)
    seg_ref[b, :, 1:W + 1] = top
    for j in range(R):
        seg_ref[b, :, (j + 1) * WP + 1:(j + 1) * WP + 1 + W] = cen[j]
    seg_ref[b, :, (R + 1) * WP + 1:(R + 1) * WP + 1 + W] = nxt
    top_ref[...] = cen[R - 1]

    seg = seg_ref[jax.lax.rem(i + 1, 2)]
    taps = jnp.concatenate(
        [seg[:, dy * WP + dx:dy * WP + dx + M]
         for dy in range(3) for dx in range(3)], axis=0)
    acc = jnp.dot(w_ref[...], taps, preferred_element_type=jnp.float32)
    y = jnp.maximum(acc + cb_ref[...], 0.0).astype(jnp.bfloat16)
    o = jnp.dot(wcr_ref[...], y, preferred_element_type=jnp.float32)
    o = o + crb_ref[...]
    for r in range(R):
        row = o[:, r * WP:r * WP + W]
        cls_ref[:, r, :] = row[:18]
        reg_ref[:, r, :] = row[18:]


def kernel(x, conv_w, conv_b, cls_w, cls_b, reg_w, reg_b):
    x3 = x.reshape(CIN, H, W)
    wt = conv_w.transpose(0, 2, 3, 1).reshape(CMID, 9 * CIN).astype(jnp.bfloat16)
    wcr = jnp.concatenate(
        [cls_w.reshape(-1, CMID), reg_w.reshape(-1, CMID)]).astype(jnp.bfloat16)
    cb = conv_b.reshape(CMID, 1)
    crb = jnp.concatenate([cls_b, reg_b]).reshape(54, 1)

    cls_out, reg_out = pl.pallas_call(
        _rpn_body,
        grid=(T + 1,),
        in_specs=[
            pl.BlockSpec((CIN, R, W), lambda i: (0, jnp.minimum(i, T - 1), 0)),
            pl.BlockSpec((CIN, 8, W),
                         lambda i: (0, jnp.minimum((i + 1) * (R // 8), H // 8 - 1), 0)),
            pl.BlockSpec((CMID, 9 * CIN), lambda i: (0, 0)),
            pl.BlockSpec((54, CMID), lambda i: (0, 0)),
            pl.BlockSpec((CMID, 1), lambda i: (0, 0)),
            pl.BlockSpec((54, 1), lambda i: (0, 0)),
        ],
        out_specs=[
            pl.BlockSpec((18, R, W), lambda i: (0, jnp.maximum(i - 1, 0), 0)),
            pl.BlockSpec((36, R, W), lambda i: (0, jnp.maximum(i - 1, 0), 0)),
        ],
        out_shape=[
            jax.ShapeDtypeStruct((18, H, W), jnp.float32),
            jax.ShapeDtypeStruct((36, H, W), jnp.float32),
        ],
        scratch_shapes=[
            pltpu.VMEM((2, CIN, SEG), jnp.bfloat16),
            pltpu.VMEM((CIN, W), jnp.bfloat16),
        ],
        compiler_params=pltpu.CompilerParams(
            dimension_semantics=("arbitrary",)),
    )(x3, x3, wt, wcr, cb, crb)

    return (cls_out.reshape(1, 18, H, W), reg_out.reshape(1, 36, H, W))


# speedup vs baseline: 1.1113x; 1.1113x over previous
import jax
import jax.numpy as jnp
from jax.experimental import pallas as pl
from jax.experimental.pallas import tpu as pltpu

H = 224
W = 224
WP = 256
CIN = 192
CMID = 256
R = 16
T2 = H // (2 * R)
M = R * WP
SEG = (R + 2) * WP + 128


def _rpn_body(a_ref, b_ref, w_ref, wcr_ref, cb_ref, crb_ref,
              cls_ref, reg_ref, seg0_ref, seg1_ref, top_ref):
    k = pl.program_id(0)

    cen = jnp.transpose(a_ref[...].astype(jnp.bfloat16), (1, 0, 2))
    nxt = b_ref[:, 0, :].astype(jnp.bfloat16)
    top = jnp.where(k == 0, jnp.zeros_like(top_ref), top_ref[...])
    nxt = jnp.where(k == T2 - 1, jnp.zeros_like(nxt), nxt)

    @pl.when(k == 0)
    def _():
        seg0_ref[...] = jnp.zeros((CIN, SEG), jnp.bfloat16)
        seg1_ref[...] = jnp.zeros((CIN, SEG), jnp.bfloat16)

    def restride(seg_ref, rows):
        for j, row in enumerate(rows):
            seg_ref[:, j * WP + 1:j * WP + 1 + W] = row

    def compute(seg_ref, half):
        seg = seg_ref[...]
        taps = jnp.concatenate(
            [seg[:, dy * WP + dx:dy * WP + dx + M]
             for dy in range(3) for dx in range(3)], axis=0)
        acc = jnp.dot(w_ref[...], taps, preferred_element_type=jnp.float32)
        y = jnp.maximum(acc + cb_ref[...], 0.0).astype(jnp.bfloat16)
        o = jnp.dot(wcr_ref[...], y, preferred_element_type=jnp.float32)
        o = o + crb_ref[...]
        for r in range(R):
            row = o[:, r * WP:r * WP + W]
            rr = half * R + r
            cls_ref[:, rr, :] = row[:18]
            reg_ref[:, rr, :] = row[18:]

    restride(seg0_ref, [top] + [cen[j] for j in range(R + 1)])
    restride(seg1_ref, [cen[j] for j in range(R - 1, 2 * R)] + [nxt])
    compute(seg0_ref, 0)
    compute(seg1_ref, 1)

    top_ref[...] = cen[2 * R - 1]


def kernel(x, conv_w, conv_b, cls_w, cls_b, reg_w, reg_b):
    x3 = x.reshape(CIN, H, W)
    wt = conv_w.transpose(0, 2, 3, 1).reshape(CMID, 9 * CIN).astype(jnp.bfloat16)
    wcr = jnp.concatenate(
        [cls_w.reshape(-1, CMID), reg_w.reshape(-1, CMID)]).astype(jnp.bfloat16)
    cb = conv_b.reshape(CMID, 1)
    crb = jnp.concatenate([cls_b, reg_b]).reshape(54, 1)

    cls_out, reg_out = pl.pallas_call(
        _rpn_body,
        grid=(T2,),
        in_specs=[
            pl.BlockSpec((CIN, 2 * R, W), lambda k: (0, k, 0)),
            pl.BlockSpec((CIN, 8, W),
                         lambda k: (0, jnp.minimum((k + 1) * (2 * R // 8), H // 8 - 1), 0)),
            pl.BlockSpec((CMID, 9 * CIN), lambda k: (0, 0)),
            pl.BlockSpec((54, CMID), lambda k: (0, 0)),
            pl.BlockSpec((CMID, 1), lambda k: (0, 0)),
            pl.BlockSpec((54, 1), lambda k: (0, 0)),
        ],
        out_specs=[
            pl.BlockSpec((18, 2 * R, W), lambda k: (0, k, 0)),
            pl.BlockSpec((36, 2 * R, W), lambda k: (0, k, 0)),
        ],
        out_shape=[
            jax.ShapeDtypeStruct((18, H, W), jnp.float32),
            jax.ShapeDtypeStruct((36, H, W), jnp.float32),
        ],
        scratch_shapes=[
            pltpu.VMEM((CIN, SEG), jnp.bfloat16),
            pltpu.VMEM((CIN, SEG), jnp.bfloat16),
            pltpu.VMEM((CIN, W), jnp.bfloat16),
        ],
        compiler_params=pltpu.CompilerParams(
            dimension_semantics=("arbitrary",)),
    )(x3, x3, wt, wcr, cb, crb)

    return (cls_out.reshape(1, 18, H, W), reg_out.reshape(1, 36, H, W))


# speedup vs baseline: 1.1149x; 1.0033x over previous
import jax
import jax.numpy as jnp
from jax.experimental import pallas as pl
from jax.experimental.pallas import tpu as pltpu

H = 224
W = 224
WP = 256
CIN = 192
CMID = 256
R = 16
T2 = H // (2 * R)
M = R * WP
SEG = (R + 2) * WP + 128


def _rpn_body(a_ref, b_ref, w_ref, wcr_ref, cb_ref, crb_ref,
              cls_ref, reg_ref, seg0_ref, seg1_ref, top_ref):
    k = pl.program_id(0)

    cen = jnp.transpose(a_ref[...].astype(jnp.bfloat16), (1, 0, 2))
    nxt = b_ref[:, 0, :].astype(jnp.bfloat16)
    top = jnp.where(k == 0, jnp.zeros_like(top_ref), top_ref[...])
    nxt = jnp.where(k == T2 - 1, jnp.zeros_like(nxt), nxt)

    @pl.when(k == 0)
    def _():
        seg0_ref[...] = jnp.zeros((CIN, SEG), jnp.bfloat16)
        seg1_ref[...] = jnp.zeros((CIN, SEG), jnp.bfloat16)

    def restride(seg_ref, rows):
        for j, row in enumerate(rows):
            seg_ref[:, j * WP + 1:j * WP + 1 + W] = row

    def compute(seg_ref, half):
        seg = seg_ref[...]
        taps = jnp.concatenate(
            [seg[:, dy * WP + dx:dy * WP + dx + M]
             for dy in range(3) for dx in range(3)], axis=0)
        acc = jnp.dot(w_ref[...], taps, preferred_element_type=jnp.float32)
        y = jnp.maximum(acc + cb_ref[...], 0.0).astype(jnp.bfloat16)
        o = jnp.dot(wcr_ref[...], y, preferred_element_type=jnp.float32)
        o = o + crb_ref[...]
        for r in range(R):
            row = o[:, r * WP:r * WP + W]
            rr = half * R + r
            cls_ref[:, rr, :] = row[:18]
            reg_ref[:, rr, :] = row[18:]

    restride(seg0_ref, [top] + [cen[j] for j in range(R + 1)])
    restride(seg1_ref, [cen[j] for j in range(R - 1, 2 * R)] + [nxt])
    compute(seg0_ref, 0)
    compute(seg1_ref, 1)

    top_ref[...] = cen[2 * R - 1]


def kernel(x, conv_w, conv_b, cls_w, cls_b, reg_w, reg_b):
    x3 = x.reshape(CIN, H, W)
    wt = conv_w.astype(jnp.bfloat16).transpose(0, 2, 3, 1).reshape(CMID, 9 * CIN)
    wcr = jnp.concatenate(
        [cls_w.reshape(-1, CMID), reg_w.reshape(-1, CMID)]).astype(jnp.bfloat16)
    cb = conv_b.reshape(CMID, 1)
    crb = jnp.concatenate([cls_b, reg_b]).reshape(54, 1)

    cls_out, reg_out = pl.pallas_call(
        _rpn_body,
        grid=(T2,),
        in_specs=[
            pl.BlockSpec((CIN, 2 * R, W), lambda k: (0, k, 0)),
            pl.BlockSpec((CIN, 8, W),
                         lambda k: (0, jnp.minimum((k + 1) * (2 * R // 8), H // 8 - 1), 0)),
            pl.BlockSpec((CMID, 9 * CIN), lambda k: (0, 0)),
            pl.BlockSpec((54, CMID), lambda k: (0, 0)),
            pl.BlockSpec((CMID, 1), lambda k: (0, 0)),
            pl.BlockSpec((54, 1), lambda k: (0, 0)),
        ],
        out_specs=[
            pl.BlockSpec((18, 2 * R, W), lambda k: (0, k, 0)),
            pl.BlockSpec((36, 2 * R, W), lambda k: (0, k, 0)),
        ],
        out_shape=[
            jax.ShapeDtypeStruct((18, H, W), jnp.float32),
            jax.ShapeDtypeStruct((36, H, W), jnp.float32),
        ],
        scratch_shapes=[
            pltpu.VMEM((CIN, SEG), jnp.bfloat16),
            pltpu.VMEM((CIN, SEG), jnp.bfloat16),
            pltpu.VMEM((CIN, W), jnp.bfloat16),
        ],
        compiler_params=pltpu.CompilerParams(
            dimension_semantics=("arbitrary",)),
    )(x3, x3, wt, wcr, cb, crb)

    return (cls_out.reshape(1, 18, H, W), reg_out.reshape(1, 36, H, W))
